# Initial kernel scaffold; baseline (speedup 1.0000x reference)
import functools

import jax
import jax.numpy as jnp
from jax import lax
from jax.experimental import pallas as pl
from jax.experimental.pallas import tpu as pltpu
from jax.experimental.pallas import tpu_sc as plsc

N = 10000
E = 320000
IN_CH = 128
HID = 256
H1 = 128
H2 = 64
EPS = 1e-5

NPAD = 10240
CHUNK = 128
ROWS_PER_TILE = NPAD // 16

E_PER_TILE_1 = E // 32
NCH1 = 79
E_PAD_1 = NCH1 * CHUNK

E_PER_TILE_2 = E // 16
NCH2 = 158
E_PAD_2 = NCH2 * CHUNK

_mesh = plsc.VectorSubcoreMesh(core_axis_name="c", subcore_axis_name="s")


def _agg1_body(x_hbm, src_hbm, dst_hbm, z128_hbm, z8_hbm, o8_hbm,
               agg_out, cnt_out,
               src_v, dst_v, rows_v, ones_v, agg_sh, cnt_sh, sem):
    c = lax.axis_index("c")
    s = lax.axis_index("s")
    w = c * 16 + s
    base = s * ROWS_PER_TILE

    def zero_loop(i, carry):
        pltpu.sync_copy(z128_hbm, agg_sh.at[pl.ds(base + i * CHUNK, CHUNK)])
        pltpu.sync_copy(z8_hbm, cnt_sh.at[pl.ds(base + i * CHUNK, CHUNK)])
        return carry
    lax.fori_loop(0, ROWS_PER_TILE // CHUNK, zero_loop, 0)

    pltpu.sync_copy(o8_hbm, ones_v)
    pltpu.sync_copy(src_hbm.at[w], src_v)
    pltpu.sync_copy(dst_hbm.at[w], dst_v)
    plsc.subcore_barrier()

    def edge_loop(j, carry):
        pltpu.async_copy(x_hbm.at[src_v.at[j]], rows_v, sem).wait()
        pltpu.sync_copy(rows_v, agg_sh.at[dst_v.at[j]], add=True)
        pltpu.sync_copy(ones_v, cnt_sh.at[dst_v.at[j]], add=True)
        return carry
    lax.fori_loop(0, NCH1, edge_loop, 0)
    plsc.subcore_barrier()

    pltpu.sync_copy(agg_sh.at[pl.ds(base, ROWS_PER_TILE)],
                    agg_out.at[c, pl.ds(base, ROWS_PER_TILE)])
    pltpu.sync_copy(cnt_sh.at[pl.ds(base, ROWS_PER_TILE)],
                    cnt_out.at[c, pl.ds(base, ROWS_PER_TILE)])


_agg1 = functools.partial(
    pl.kernel,
    out_type=(jax.ShapeDtypeStruct((2, NPAD, IN_CH), jnp.float32),
              jax.ShapeDtypeStruct((2, NPAD, 8), jnp.float32)),
    mesh=_mesh,
    scratch_types=[
        pltpu.VMEM((NCH1, CHUNK), jnp.int32),
        pltpu.VMEM((NCH1, CHUNK), jnp.int32),
        pltpu.VMEM((CHUNK, IN_CH), jnp.float32),
        pltpu.VMEM((CHUNK, 8), jnp.float32),
        pltpu.VMEM_SHARED((NPAD, IN_CH), jnp.float32),
        pltpu.VMEM_SHARED((NPAD, 8), jnp.float32),
        pltpu.SemaphoreType.DMA,
    ],
)(_agg1_body)


def _agg2_body(h1a_hbm, h1b_hbm, src_hbm, dst_hbm, z128_hbm,
               agg_out,
               src_v, dst_v, rows_v, agg_sh, sem):
    c = lax.axis_index("c")
    s = lax.axis_index("s")
    base = s * ROWS_PER_TILE

    def zero_loop(i, carry):
        pltpu.sync_copy(z128_hbm, agg_sh.at[pl.ds(base + i * CHUNK, CHUNK)])
        return carry
    lax.fori_loop(0, ROWS_PER_TILE // CHUNK, zero_loop, 0)

    pltpu.sync_copy(src_hbm.at[s], src_v)
    pltpu.sync_copy(dst_hbm.at[s], dst_v)
    plsc.subcore_barrier()

    def edge_loop(j, carry):
        @pl.when(c == 0)
        def _():
            pltpu.async_copy(h1a_hbm.at[src_v.at[j]], rows_v, sem).wait()

        @pl.when(c == 1)
        def _():
            pltpu.async_copy(h1b_hbm.at[src_v.at[j]], rows_v, sem).wait()

        pltpu.sync_copy(rows_v, agg_sh.at[dst_v.at[j]], add=True)
        return carry
    lax.fori_loop(0, NCH2, edge_loop, 0)
    plsc.subcore_barrier()

    pltpu.sync_copy(agg_sh.at[pl.ds(base, ROWS_PER_TILE)],
                    agg_out.at[c, pl.ds(base, ROWS_PER_TILE)])


_agg2 = functools.partial(
    pl.kernel,
    out_type=jax.ShapeDtypeStruct((2, NPAD, IN_CH), jnp.float32),
    mesh=_mesh,
    scratch_types=[
        pltpu.VMEM((NCH2, CHUNK), jnp.int32),
        pltpu.VMEM((NCH2, CHUNK), jnp.int32),
        pltpu.VMEM((CHUNK, IN_CH), jnp.float32),
        pltpu.VMEM_SHARED((NPAD, IN_CH), jnp.float32),
        pltpu.SemaphoreType.DMA,
    ],
)(_agg2_body)


BLK = 1024


def _mlp1_body(aggp, cnt8, x, wl1aT, wl1bT, wr1aT, wr1bT, bl1a, bl1b,
               h1a, h1b):
    cnt = cnt8[0, :, 0:1] + cnt8[1, :, 0:1]
    rec = 1.0 / jnp.maximum(cnt, 1.0)
    agg = (aggp[0] + aggp[1]) * rec
    xv = x[...]
    h1a[...] = jnp.maximum(
        jnp.dot(agg, wl1aT[...], preferred_element_type=jnp.float32)
        + jnp.dot(xv, wr1aT[...], preferred_element_type=jnp.float32)
        + bl1a[...], 0.0)
    h1b[...] = jnp.maximum(
        jnp.dot(agg, wl1bT[...], preferred_element_type=jnp.float32)
        + jnp.dot(xv, wr1bT[...], preferred_element_type=jnp.float32)
        + bl1b[...], 0.0)


def _mlp1(aggp, cnt8, xp, wl1aT, wl1bT, wr1aT, wr1bT, bl1a, bl1b):
    grid = NPAD // BLK
    return pl.pallas_call(
        _mlp1_body,
        grid=(grid,),
        in_specs=[
            pl.BlockSpec((2, BLK, IN_CH), lambda i: (0, i, 0)),
            pl.BlockSpec((2, BLK, 8), lambda i: (0, i, 0)),
            pl.BlockSpec((BLK, IN_CH), lambda i: (i, 0)),
            pl.BlockSpec((IN_CH, 128), lambda i: (0, 0)),
            pl.BlockSpec((IN_CH, 128), lambda i: (0, 0)),
            pl.BlockSpec((IN_CH, 128), lambda i: (0, 0)),
            pl.BlockSpec((IN_CH, 128), lambda i: (0, 0)),
            pl.BlockSpec((1, 128), lambda i: (0, 0)),
            pl.BlockSpec((1, 128), lambda i: (0, 0)),
        ],
        out_specs=[
            pl.BlockSpec((BLK, 128), lambda i: (i, 0)),
            pl.BlockSpec((BLK, 128), lambda i: (i, 0)),
        ],
        out_shape=[
            jax.ShapeDtypeStruct((NPAD, 128), jnp.float32),
            jax.ShapeDtypeStruct((NPAD, 128), jnp.float32),
        ],
    )(aggp, cnt8, xp, wl1aT, wl1bT, wr1aT, wr1bT, bl1a, bl1b)


def _mlp2_body(agg2p, cnt8, h1a, h1b, wl2aT, wl2bT, wr2aT, wr2bT, bl2r,
               w1T, b1r, z1):
    cnt = cnt8[0, :, 0:1] + cnt8[1, :, 0:1]
    rec = 1.0 / jnp.maximum(cnt, 1.0)
    aA = agg2p[0] * rec
    aB = agg2p[1] * rec
    h2 = jnp.maximum(
        jnp.dot(aA, wl2aT[...], preferred_element_type=jnp.float32)
        + jnp.dot(aB, wl2bT[...], preferred_element_type=jnp.float32)
        + jnp.dot(h1a[...], wr2aT[...], preferred_element_type=jnp.float32)
        + jnp.dot(h1b[...], wr2bT[...], preferred_element_type=jnp.float32)
        + bl2r[...], 0.0)
    z1[...] = jnp.dot(h2, w1T[...], preferred_element_type=jnp.float32) + b1r[...]


def _mlp2(agg2p, cnt8, h1a, h1b, wl2aT, wl2bT, wr2aT, wr2bT, bl2r, w1T, b1r):
    grid = NPAD // BLK
    return pl.pallas_call(
        _mlp2_body,
        grid=(grid,),
        in_specs=[
            pl.BlockSpec((2, BLK, 128), lambda i: (0, i, 0)),
            pl.BlockSpec((2, BLK, 8), lambda i: (0, i, 0)),
            pl.BlockSpec((BLK, 128), lambda i: (i, 0)),
            pl.BlockSpec((BLK, 128), lambda i: (i, 0)),
            pl.BlockSpec((128, HID), lambda i: (0, 0)),
            pl.BlockSpec((128, HID), lambda i: (0, 0)),
            pl.BlockSpec((128, HID), lambda i: (0, 0)),
            pl.BlockSpec((128, HID), lambda i: (0, 0)),
            pl.BlockSpec((1, HID), lambda i: (0, 0)),
            pl.BlockSpec((HID, H1), lambda i: (0, 0)),
            pl.BlockSpec((1, H1), lambda i: (0, 0)),
        ],
        out_specs=pl.BlockSpec((BLK, H1), lambda i: (i, 0)),
        out_shape=jax.ShapeDtypeStruct((NPAD, H1), jnp.float32),
    )(agg2p, cnt8, h1a, h1b, wl2aT, wl2bT, wr2aT, wr2bT, bl2r, w1T, b1r)


def _head_body(z1, g1r, be1r, w2T, b2r, g2r, be2r, w3T, b3r, out):
    z = z1[...]
    mask = (lax.broadcasted_iota(jnp.int32, (NPAD, 1), 0) < N).astype(jnp.float32)
    inv = 1.0 / N
    mu1 = jnp.sum(z * mask, axis=0, keepdims=True) * inv
    d1 = (z - mu1) * mask
    var1 = jnp.sum(d1 * d1, axis=0, keepdims=True) * inv
    a1 = jnp.maximum(g1r[...] * (z - mu1) * lax.rsqrt(var1 + EPS) + be1r[...], 0.0)
    z2 = jnp.dot(a1, w2T[...], preferred_element_type=jnp.float32) + b2r[...]
    mu2 = jnp.sum(z2 * mask, axis=0, keepdims=True) * inv
    d2 = (z2 - mu2) * mask
    var2 = jnp.sum(d2 * d2, axis=0, keepdims=True) * inv
    a2 = jnp.maximum(g2r[...] * (z2 - mu2) * lax.rsqrt(var2 + EPS) + be2r[...], 0.0)
    out[...] = jnp.dot(a2, w3T[...], preferred_element_type=jnp.float32) + b3r[...]


def _head(z1, g1r, be1r, w2T, b2r, g2r, be2r, w3T8, b3r8):
    return pl.pallas_call(
        _head_body,
        out_shape=jax.ShapeDtypeStruct((NPAD, 8), jnp.float32),
    )(z1, g1r, be1r, w2T, b2r, g2r, be2r, w3T8, b3r8)


def kernel(x, edge_index, Wl1, bl1, Wr1, Wl2, bl2, Wr2,
           W1, b1, g1, be1, W2, b2, g2, be2, W3, b3):
    f32 = jnp.float32
    xp = jnp.pad(x, ((0, NPAD - N), (0, 0)))

    ei = edge_index.astype(jnp.int32)
    src = ei[0].reshape(32, E_PER_TILE_1)
    dst = ei[1].reshape(32, E_PER_TILE_1)
    srcp = jnp.pad(src, ((0, 0), (0, E_PAD_1 - E_PER_TILE_1)))
    dstp = jnp.pad(dst, ((0, 0), (0, E_PAD_1 - E_PER_TILE_1)), constant_values=N)
    src1 = srcp.reshape(32, NCH1, CHUNK)
    dst1 = dstp.reshape(32, NCH1, CHUNK)
    src2 = srcp.reshape(16, NCH2, CHUNK)
    dst2 = dstp.reshape(16, NCH2, CHUNK)

    z128 = jnp.zeros((CHUNK, IN_CH), f32)
    z8 = jnp.zeros((CHUNK, 8), f32)
    o8 = jnp.ones((CHUNK, 8), f32)

    aggp, cnt8 = _agg1(xp, src1, dst1, z128, z8, o8)

    wl1T = Wl1.T
    wr1T = Wr1.T
    h1a, h1b = _mlp1(aggp, cnt8, xp,
                     wl1T[:, :128], wl1T[:, 128:],
                     wr1T[:, :128], wr1T[:, 128:],
                     bl1[:128].reshape(1, 128), bl1[128:].reshape(1, 128))

    agg2p = _agg2(h1a, h1b, src2, dst2, z128)

    wl2T = Wl2.T
    wr2T = Wr2.T
    z1 = _mlp2(agg2p, cnt8, h1a, h1b,
               wl2T[:128], wl2T[128:], wr2T[:128], wr2T[128:],
               bl2.reshape(1, HID), W1.T, b1.reshape(1, H1))

    w3T8 = jnp.broadcast_to(W3.T, (H2, 8))
    b3r8 = jnp.broadcast_to(b3.reshape(1, 1), (1, 8))
    out8 = _head(z1, g1.reshape(1, H1), be1.reshape(1, H1),
                 W2.T, b2.reshape(1, H2), g2.reshape(1, H2), be2.reshape(1, H2),
                 w3T8, b3r8)
    return out8[:N, 0]


# speedup vs baseline: 3.0917x; 3.0917x over previous
import functools

import jax
import jax.numpy as jnp
from jax import lax
from jax.experimental import pallas as pl
from jax.experimental.pallas import tpu as pltpu
from jax.experimental.pallas import tpu_sc as plsc

N = 10000
E = 320000
IN_CH = 128
HID = 256
H1 = 128
H2 = 64
EPS = 1e-5

NPAD = 10240
CHUNK = 128
CNTW = 16
AUGW = IN_CH + CNTW
ROWS_PER_TILE = NPAD // 16

E_PER_TILE_1 = E // 32
NCH1 = 80
E_PAD_1 = NCH1 * CHUNK

E_PER_TILE_2 = E // 16
NCH2 = 160
E_PAD_2 = NCH2 * CHUNK

G = 8

@functools.lru_cache(maxsize=None)
def _mesh():
    return plsc.VectorSubcoreMesh(core_axis_name="c", subcore_axis_name="s")


def _fill_iota(idx_v, off):
    for k in range(CHUNK // 16):
        idx_v[pl.ds(k * 16, 16)] = lax.iota(jnp.int32, 16) + (off + k * 16)


def _agg1_body(x_hbm, src_hbm, dst_hbm, z128_hbm,
               agg_out,
               src_v, dst_v, rows_v, agg_sh, sem):
    c = lax.axis_index("c")
    s = lax.axis_index("s")
    w = c * 16 + s
    base = s * ROWS_PER_TILE

    pltpu.sync_copy(z128_hbm, rows_v)

    def zero_loop(i, carry):
        _fill_iota(dst_v, base + i * CHUNK)
        pltpu.sync_copy(rows_v, agg_sh.at[dst_v])
        return carry
    lax.fori_loop(0, ROWS_PER_TILE // CHUNK, zero_loop, 0)

    plsc.subcore_barrier()

    def edge_loop(j, carry):
        pltpu.sync_copy(src_hbm.at[w, j], src_v)
        pltpu.sync_copy(dst_hbm.at[w, j], dst_v)
        pltpu.async_copy(x_hbm.at[src_v], rows_v, sem).wait()
        pltpu.sync_copy(rows_v, agg_sh.at[dst_v], add=True)
        return carry
    lax.fori_loop(0, NCH1, edge_loop, 0)
    plsc.subcore_barrier()

    def flush_loop(i, carry):
        off = base + i * CHUNK
        _fill_iota(dst_v, off)
        pltpu.async_copy(agg_sh.at[dst_v], rows_v, sem).wait()
        pltpu.sync_copy(rows_v, agg_out.at[c, pl.ds(off, CHUNK)])
        return carry
    lax.fori_loop(0, ROWS_PER_TILE // CHUNK, flush_loop, 0)


@functools.lru_cache(maxsize=None)
def _agg1():
    return pl.kernel(
        _agg1_body,
        out_type=jax.ShapeDtypeStruct((2, NPAD, IN_CH), jnp.float32),
        mesh=_mesh(),
        scratch_types=[
            pltpu.VMEM((CHUNK,), jnp.int32),
            pltpu.VMEM((CHUNK,), jnp.int32),
            pltpu.VMEM((CHUNK, IN_CH), jnp.float32),
            pltpu.VMEM_SHARED((NPAD, IN_CH), jnp.float32),
            pltpu.SemaphoreType.DMA,
        ],
    )


def _cnt_body(dst_hbm, z128_hbm, o128_hbm,
              cnt_out,
              dst_v, rows_v, ones_v, cnt_sh, sem):
    c = lax.axis_index("c")
    s = lax.axis_index("s")
    w = c * 16 + s
    base = s * ROWS_PER_TILE

    pltpu.sync_copy(z128_hbm, rows_v)
    pltpu.sync_copy(o128_hbm, ones_v)

    def zero_loop(i, carry):
        _fill_iota(dst_v, base + i * CHUNK)
        pltpu.sync_copy(rows_v, cnt_sh.at[dst_v])
        return carry
    lax.fori_loop(0, ROWS_PER_TILE // CHUNK, zero_loop, 0)

    plsc.subcore_barrier()

    def edge_loop(j, carry):
        pltpu.sync_copy(dst_hbm.at[w, j], dst_v)
        pltpu.sync_copy(ones_v, cnt_sh.at[dst_v], add=True)
        return carry
    lax.fori_loop(0, NCH1, edge_loop, 0)
    plsc.subcore_barrier()

    def flush_loop(i, carry):
        off = base + i * CHUNK
        _fill_iota(dst_v, off)
        pltpu.async_copy(cnt_sh.at[dst_v], rows_v, sem).wait()
        pltpu.sync_copy(rows_v, cnt_out.at[c, pl.ds(off, CHUNK)])
        return carry
    lax.fori_loop(0, ROWS_PER_TILE // CHUNK, flush_loop, 0)


@functools.lru_cache(maxsize=None)
def _cnt():
    return pl.kernel(
        _cnt_body,
        out_type=jax.ShapeDtypeStruct((2, NPAD, IN_CH), jnp.float32),
        mesh=_mesh(),
        scratch_types=[
            pltpu.VMEM((CHUNK,), jnp.int32),
            pltpu.VMEM((CHUNK, IN_CH), jnp.float32),
            pltpu.VMEM((CHUNK, IN_CH), jnp.float32),
            pltpu.VMEM_SHARED((NPAD, IN_CH), jnp.float32),
            pltpu.SemaphoreType.DMA,
        ],
    )


def _agg2_body(h1a_hbm, h1b_hbm, src_hbm, dst_hbm, z128_hbm,
               agg_out,
               src_v, dst_v, rows_v, agg_sh, sem):
    c = lax.axis_index("c")
    s = lax.axis_index("s")
    base = s * ROWS_PER_TILE

    pltpu.sync_copy(z128_hbm, rows_v)

    def zero_loop(i, carry):
        _fill_iota(dst_v, base + i * CHUNK)
        pltpu.sync_copy(rows_v, agg_sh.at[dst_v])
        return carry
    lax.fori_loop(0, ROWS_PER_TILE // CHUNK, zero_loop, 0)

    plsc.subcore_barrier()

    def edge_loop(j, carry):
        pltpu.sync_copy(src_hbm.at[s, j], src_v)
        pltpu.sync_copy(dst_hbm.at[s, j], dst_v)

        @pl.when(c == 0)
        def _():
            pltpu.async_copy(h1a_hbm.at[src_v], rows_v, sem).wait()

        @pl.when(c == 1)
        def _():
            pltpu.async_copy(h1b_hbm.at[src_v], rows_v, sem).wait()

        pltpu.sync_copy(rows_v, agg_sh.at[dst_v], add=True)
        return carry
    lax.fori_loop(0, NCH2, edge_loop, 0)
    plsc.subcore_barrier()

    def flush_loop(i, carry):
        off = base + i * CHUNK
        _fill_iota(dst_v, off)
        pltpu.async_copy(agg_sh.at[dst_v], rows_v, sem).wait()
        pltpu.sync_copy(rows_v, agg_out.at[c, pl.ds(off, CHUNK)])
        return carry
    lax.fori_loop(0, ROWS_PER_TILE // CHUNK, flush_loop, 0)


@functools.lru_cache(maxsize=None)
def _agg2():
    return pl.kernel(
        _agg2_body,
        out_type=jax.ShapeDtypeStruct((2, NPAD, IN_CH), jnp.float32),
        mesh=_mesh(),
        scratch_types=[
            pltpu.VMEM((CHUNK,), jnp.int32),
            pltpu.VMEM((CHUNK,), jnp.int32),
            pltpu.VMEM((CHUNK, IN_CH), jnp.float32),
            pltpu.VMEM_SHARED((NPAD, IN_CH), jnp.float32),
            pltpu.SemaphoreType.DMA,
        ],
    )


BLK = 1024


def _mlp1_body(aggp, cntp, x, wl1aT, wl1bT, wr1aT, wr1bT, bl1a, bl1b,
               h1a, h1b, rec8):
    cnt = cntp[0, :, 0:1] + cntp[1, :, 0:1]
    rec = 1.0 / jnp.maximum(cnt, 1.0)
    agg = (aggp[0] + aggp[1]) * rec
    xv = x[...]
    rec8[...] = jnp.broadcast_to(rec, (rec.shape[0], 8))
    h1a[...] = jnp.maximum(
        jnp.dot(agg, wl1aT[...], preferred_element_type=jnp.float32)
        + jnp.dot(xv, wr1aT[...], preferred_element_type=jnp.float32)
        + bl1a[...], 0.0)
    h1b[...] = jnp.maximum(
        jnp.dot(agg, wl1bT[...], preferred_element_type=jnp.float32)
        + jnp.dot(xv, wr1bT[...], preferred_element_type=jnp.float32)
        + bl1b[...], 0.0)


def _mlp1(aggp, cntp, xp, wl1aT, wl1bT, wr1aT, wr1bT, bl1a, bl1b):
    grid = NPAD // BLK
    return pl.pallas_call(
        _mlp1_body,
        grid=(grid,),
        in_specs=[
            pl.BlockSpec((2, BLK, IN_CH), lambda i: (0, i, 0)),
            pl.BlockSpec((2, BLK, IN_CH), lambda i: (0, i, 0)),
            pl.BlockSpec((BLK, IN_CH), lambda i: (i, 0)),
            pl.BlockSpec((IN_CH, 128), lambda i: (0, 0)),
            pl.BlockSpec((IN_CH, 128), lambda i: (0, 0)),
            pl.BlockSpec((IN_CH, 128), lambda i: (0, 0)),
            pl.BlockSpec((IN_CH, 128), lambda i: (0, 0)),
            pl.BlockSpec((1, 128), lambda i: (0, 0)),
            pl.BlockSpec((1, 128), lambda i: (0, 0)),
        ],
        out_specs=[
            pl.BlockSpec((BLK, 128), lambda i: (i, 0)),
            pl.BlockSpec((BLK, 128), lambda i: (i, 0)),
            pl.BlockSpec((BLK, 8), lambda i: (i, 0)),
        ],
        out_shape=[
            jax.ShapeDtypeStruct((NPAD, 128), jnp.float32),
            jax.ShapeDtypeStruct((NPAD, 128), jnp.float32),
            jax.ShapeDtypeStruct((NPAD, 8), jnp.float32),
        ],
    )(aggp, cntp, xp, wl1aT, wl1bT, wr1aT, wr1bT, bl1a, bl1b)


def _mlp2_body(agg2p, rec8, h1a, h1b, wl2aT, wl2bT, wr2aT, wr2bT, bl2r,
               w1T, b1r, z1):
    rec = rec8[:, 0:1]
    aA = agg2p[0] * rec
    aB = agg2p[1] * rec
    h2 = jnp.maximum(
        jnp.dot(aA, wl2aT[...], preferred_element_type=jnp.float32)
        + jnp.dot(aB, wl2bT[...], preferred_element_type=jnp.float32)
        + jnp.dot(h1a[...], wr2aT[...], preferred_element_type=jnp.float32)
        + jnp.dot(h1b[...], wr2bT[...], preferred_element_type=jnp.float32)
        + bl2r[...], 0.0)
    z1[...] = jnp.dot(h2, w1T[...], preferred_element_type=jnp.float32) + b1r[...]


def _mlp2(agg2p, rec8, h1a, h1b, wl2aT, wl2bT, wr2aT, wr2bT, bl2r, w1T, b1r):
    grid = NPAD // BLK
    return pl.pallas_call(
        _mlp2_body,
        grid=(grid,),
        in_specs=[
            pl.BlockSpec((2, BLK, 128), lambda i: (0, i, 0)),
            pl.BlockSpec((BLK, 8), lambda i: (i, 0)),
            pl.BlockSpec((BLK, 128), lambda i: (i, 0)),
            pl.BlockSpec((BLK, 128), lambda i: (i, 0)),
            pl.BlockSpec((128, HID), lambda i: (0, 0)),
            pl.BlockSpec((128, HID), lambda i: (0, 0)),
            pl.BlockSpec((128, HID), lambda i: (0, 0)),
            pl.BlockSpec((128, HID), lambda i: (0, 0)),
            pl.BlockSpec((1, HID), lambda i: (0, 0)),
            pl.BlockSpec((HID, H1), lambda i: (0, 0)),
            pl.BlockSpec((1, H1), lambda i: (0, 0)),
        ],
        out_specs=pl.BlockSpec((BLK, H1), lambda i: (i, 0)),
        out_shape=jax.ShapeDtypeStruct((NPAD, H1), jnp.float32),
    )(agg2p, rec8, h1a, h1b, wl2aT, wl2bT, wr2aT, wr2bT, bl2r, w1T, b1r)


def _head_body(z1, g1r, be1r, w2T, b2r, g2r, be2r, w3T, b3r, out):
    z = z1[...]
    mask = (lax.broadcasted_iota(jnp.int32, (NPAD, 1), 0) < N).astype(jnp.float32)
    inv = 1.0 / N
    mu1 = jnp.sum(z * mask, axis=0, keepdims=True) * inv
    d1 = (z - mu1) * mask
    var1 = jnp.sum(d1 * d1, axis=0, keepdims=True) * inv
    a1 = jnp.maximum(g1r[...] * (z - mu1) * lax.rsqrt(var1 + EPS) + be1r[...], 0.0)
    z2 = jnp.dot(a1, w2T[...], preferred_element_type=jnp.float32) + b2r[...]
    mu2 = jnp.sum(z2 * mask, axis=0, keepdims=True) * inv
    d2 = (z2 - mu2) * mask
    var2 = jnp.sum(d2 * d2, axis=0, keepdims=True) * inv
    a2 = jnp.maximum(g2r[...] * (z2 - mu2) * lax.rsqrt(var2 + EPS) + be2r[...], 0.0)
    out[...] = jnp.dot(a2, w3T[...], preferred_element_type=jnp.float32) + b3r[...]


def _head(z1, g1r, be1r, w2T, b2r, g2r, be2r, w3T8, b3r8):
    return pl.pallas_call(
        _head_body,
        out_shape=jax.ShapeDtypeStruct((NPAD, 8), jnp.float32),
    )(z1, g1r, be1r, w2T, b2r, g2r, be2r, w3T8, b3r8)


def kernel(x, edge_index, Wl1, bl1, Wr1, Wl2, bl2, Wr2,
           W1, b1, g1, be1, W2, b2, g2, be2, W3, b3):
    f32 = jnp.float32
    xp = jnp.pad(x, ((0, NPAD - N), (0, 0)))

    ei = edge_index.astype(jnp.int32)
    src = ei[0].reshape(32, E_PER_TILE_1)
    dst = ei[1].reshape(32, E_PER_TILE_1)
    srcp = jnp.pad(src, ((0, 0), (0, E_PAD_1 - E_PER_TILE_1)))
    dstp = jnp.pad(dst, ((0, 0), (0, E_PAD_1 - E_PER_TILE_1)), constant_values=N)
    src1 = srcp.reshape(32, NCH1, CHUNK)
    dst1 = dstp.reshape(32, NCH1, CHUNK)
    src2 = srcp.reshape(16, NCH2, CHUNK)
    dst2 = dstp.reshape(16, NCH2, CHUNK)
    z128 = jnp.zeros((CHUNK, IN_CH), f32)
    o128 = jnp.ones((CHUNK, IN_CH), f32)

    cntp = _cnt()(dst1, z128, o128)
    aggp = _agg1()(xp, src1, dst1, z128)

    wl1T = Wl1.T
    wr1T = Wr1.T
    h1a, h1b, rec8 = _mlp1(aggp, cntp, xp,
                           wl1T[:, :128], wl1T[:, 128:],
                           wr1T[:, :128], wr1T[:, 128:],
                           bl1[:128].reshape(1, 128), bl1[128:].reshape(1, 128))

    agg2p = _agg2()(h1a, h1b, src2, dst2, z128)

    wl2T = Wl2.T
    wr2T = Wr2.T
    z1 = _mlp2(agg2p, rec8, h1a, h1b,
               wl2T[:128], wl2T[128:], wr2T[:128], wr2T[128:],
               bl2.reshape(1, HID), W1.T, b1.reshape(1, H1))

    w3T8 = jnp.broadcast_to(W3.T, (H2, 8))
    b3r8 = jnp.broadcast_to(b3.reshape(1, 1), (1, 8))
    out8 = _head(z1, g1.reshape(1, H1), be1.reshape(1, H1),
                 W2.T, b2.reshape(1, H2), g2.reshape(1, H2), be2.reshape(1, H2),
                 w3T8, b3r8)
    return out8[:N, 0]


# speedup vs baseline: 3.2551x; 1.0529x over previous
import functools

import jax
import jax.numpy as jnp
from jax import lax
from jax.experimental import pallas as pl
from jax.experimental.pallas import tpu as pltpu
from jax.experimental.pallas import tpu_sc as plsc

N = 10000
E = 320000
IN_CH = 128
HID = 256
H1 = 128
H2 = 64
EPS = 1e-5

NPAD = 10240
CHUNK = 128
CNTW = 16
AUGW = IN_CH + CNTW
ROWS_PER_TILE = NPAD // 16

E_PER_TILE_1 = E // 32
NCH1 = 80
E_PAD_1 = NCH1 * CHUNK

E_PER_TILE_2 = E // 16
NCH2 = 160
E_PAD_2 = NCH2 * CHUNK

G = 8

@functools.lru_cache(maxsize=None)
def _mesh():
    return plsc.VectorSubcoreMesh(core_axis_name="c", subcore_axis_name="s")


def _fill_iota(idx_v, off):
    for k in range(CHUNK // 16):
        idx_v[pl.ds(k * 16, 16)] = lax.iota(jnp.int32, 16) + (off + k * 16)


def _agg1_body(x_hbm, src_hbm, dst_hbm, z128_hbm,
               agg_out,
               src_v, dst_v, dst_v2, rows_v, rows_v2, agg_sh, sem, sem2):
    c = lax.axis_index("c")
    s = lax.axis_index("s")
    w = c * 16 + s
    base = s * ROWS_PER_TILE

    pltpu.sync_copy(z128_hbm, rows_v)

    def zero_loop(i, carry):
        _fill_iota(dst_v, base + i * CHUNK)
        pltpu.sync_copy(rows_v, agg_sh.at[dst_v])
        return carry
    lax.fori_loop(0, ROWS_PER_TILE // CHUNK, zero_loop, 0)

    plsc.subcore_barrier()

    def edge_loop(p, carry):
        j0 = 2 * p
        pltpu.sync_copy(src_hbm.at[w, j0], src_v)
        pltpu.sync_copy(dst_hbm.at[w, j0], dst_v)
        pltpu.async_copy(x_hbm.at[src_v], rows_v, sem).wait()
        d0 = pltpu.async_copy(rows_v, agg_sh.at[dst_v], sem2, add=True)
        pltpu.sync_copy(src_hbm.at[w, j0 + 1], src_v)
        pltpu.sync_copy(dst_hbm.at[w, j0 + 1], dst_v2)
        pltpu.async_copy(x_hbm.at[src_v], rows_v2, sem).wait()
        d0.wait()
        pltpu.sync_copy(rows_v2, agg_sh.at[dst_v2], add=True)
        return carry
    lax.fori_loop(0, NCH1 // 2, edge_loop, 0)
    plsc.subcore_barrier()

    def flush_loop(i, carry):
        off = base + i * CHUNK
        _fill_iota(dst_v, off)
        pltpu.async_copy(agg_sh.at[dst_v], rows_v, sem).wait()
        pltpu.sync_copy(rows_v, agg_out.at[c, pl.ds(off, CHUNK)])
        return carry
    lax.fori_loop(0, ROWS_PER_TILE // CHUNK, flush_loop, 0)


@functools.lru_cache(maxsize=None)
def _agg1():
    return pl.kernel(
        _agg1_body,
        out_type=jax.ShapeDtypeStruct((2, NPAD, IN_CH), jnp.float32),
        mesh=_mesh(),
        scratch_types=[
            pltpu.VMEM((CHUNK,), jnp.int32),
            pltpu.VMEM((CHUNK,), jnp.int32),
            pltpu.VMEM((CHUNK,), jnp.int32),
            pltpu.VMEM((CHUNK, IN_CH), jnp.float32),
            pltpu.VMEM((CHUNK, IN_CH), jnp.float32),
            pltpu.VMEM_SHARED((NPAD, IN_CH), jnp.float32),
            pltpu.SemaphoreType.DMA,
            pltpu.SemaphoreType.DMA,
        ],
    )


def _cnt_body(dst_hbm, z128_hbm, o128_hbm,
              cnt_out,
              dst_v, rows_v, ones_v, cnt_sh, sem):
    c = lax.axis_index("c")
    s = lax.axis_index("s")
    w = c * 16 + s
    base = s * ROWS_PER_TILE

    pltpu.sync_copy(z128_hbm, rows_v)
    pltpu.sync_copy(o128_hbm, ones_v)

    def zero_loop(i, carry):
        _fill_iota(dst_v, base + i * CHUNK)
        pltpu.sync_copy(rows_v, cnt_sh.at[dst_v])
        return carry
    lax.fori_loop(0, ROWS_PER_TILE // CHUNK, zero_loop, 0)

    plsc.subcore_barrier()

    def edge_loop(j, carry):
        pltpu.sync_copy(dst_hbm.at[w, j], dst_v)
        pltpu.sync_copy(ones_v, cnt_sh.at[dst_v], add=True)
        return carry
    lax.fori_loop(0, NCH1, edge_loop, 0)
    plsc.subcore_barrier()

    def flush_loop(i, carry):
        off = base + i * CHUNK
        _fill_iota(dst_v, off)
        pltpu.async_copy(cnt_sh.at[dst_v], rows_v, sem).wait()
        pltpu.sync_copy(rows_v, cnt_out.at[c, pl.ds(off, CHUNK)])
        return carry
    lax.fori_loop(0, ROWS_PER_TILE // CHUNK, flush_loop, 0)


@functools.lru_cache(maxsize=None)
def _cnt():
    return pl.kernel(
        _cnt_body,
        out_type=jax.ShapeDtypeStruct((2, NPAD, IN_CH), jnp.float32),
        mesh=_mesh(),
        scratch_types=[
            pltpu.VMEM((CHUNK,), jnp.int32),
            pltpu.VMEM((CHUNK, IN_CH), jnp.float32),
            pltpu.VMEM((CHUNK, IN_CH), jnp.float32),
            pltpu.VMEM_SHARED((NPAD, IN_CH), jnp.float32),
            pltpu.SemaphoreType.DMA,
        ],
    )


def _agg2_body(h1a_hbm, h1b_hbm, src_hbm, dst_hbm, z128_hbm,
               agg_out,
               src_v, dst_v, dst_v2, rows_v, rows_v2, agg_sh, sem, sem2):
    c = lax.axis_index("c")
    s = lax.axis_index("s")
    base = s * ROWS_PER_TILE

    pltpu.sync_copy(z128_hbm, rows_v)

    def zero_loop(i, carry):
        _fill_iota(dst_v, base + i * CHUNK)
        pltpu.sync_copy(rows_v, agg_sh.at[dst_v])
        return carry
    lax.fori_loop(0, ROWS_PER_TILE // CHUNK, zero_loop, 0)

    plsc.subcore_barrier()

    def edge_loop(p, carry):
        j0 = 2 * p
        pltpu.sync_copy(src_hbm.at[s, j0], src_v)
        pltpu.sync_copy(dst_hbm.at[s, j0], dst_v)

        @pl.when(c == 0)
        def _():
            pltpu.async_copy(h1a_hbm.at[src_v], rows_v, sem).wait()

        @pl.when(c == 1)
        def _():
            pltpu.async_copy(h1b_hbm.at[src_v], rows_v, sem).wait()

        d0 = pltpu.async_copy(rows_v, agg_sh.at[dst_v], sem2, add=True)
        pltpu.sync_copy(src_hbm.at[s, j0 + 1], src_v)
        pltpu.sync_copy(dst_hbm.at[s, j0 + 1], dst_v2)

        @pl.when(c == 0)
        def _():
            pltpu.async_copy(h1a_hbm.at[src_v], rows_v2, sem).wait()

        @pl.when(c == 1)
        def _():
            pltpu.async_copy(h1b_hbm.at[src_v], rows_v2, sem).wait()

        d0.wait()
        pltpu.sync_copy(rows_v2, agg_sh.at[dst_v2], add=True)
        return carry
    lax.fori_loop(0, NCH2 // 2, edge_loop, 0)
    plsc.subcore_barrier()

    def flush_loop(i, carry):
        off = base + i * CHUNK
        _fill_iota(dst_v, off)
        pltpu.async_copy(agg_sh.at[dst_v], rows_v, sem).wait()
        pltpu.sync_copy(rows_v, agg_out.at[c, pl.ds(off, CHUNK)])
        return carry
    lax.fori_loop(0, ROWS_PER_TILE // CHUNK, flush_loop, 0)


@functools.lru_cache(maxsize=None)
def _agg2():
    return pl.kernel(
        _agg2_body,
        out_type=jax.ShapeDtypeStruct((2, NPAD, IN_CH), jnp.float32),
        mesh=_mesh(),
        scratch_types=[
            pltpu.VMEM((CHUNK,), jnp.int32),
            pltpu.VMEM((CHUNK,), jnp.int32),
            pltpu.VMEM((CHUNK,), jnp.int32),
            pltpu.VMEM((CHUNK, IN_CH), jnp.float32),
            pltpu.VMEM((CHUNK, IN_CH), jnp.float32),
            pltpu.VMEM_SHARED((NPAD, IN_CH), jnp.float32),
            pltpu.SemaphoreType.DMA,
            pltpu.SemaphoreType.DMA,
        ],
    )


BLK = 1024


def _mlp1_body(aggp, cntp, x, wl1aT, wl1bT, wr1aT, wr1bT, bl1a, bl1b,
               h1a, h1b, rec8):
    cnt = cntp[0, :, 0:1] + cntp[1, :, 0:1]
    rec = 1.0 / jnp.maximum(cnt, 1.0)
    agg = (aggp[0] + aggp[1]) * rec
    xv = x[...]
    rec8[...] = jnp.broadcast_to(rec, (rec.shape[0], 8))
    h1a[...] = jnp.maximum(
        jnp.dot(agg, wl1aT[...], preferred_element_type=jnp.float32)
        + jnp.dot(xv, wr1aT[...], preferred_element_type=jnp.float32)
        + bl1a[...], 0.0)
    h1b[...] = jnp.maximum(
        jnp.dot(agg, wl1bT[...], preferred_element_type=jnp.float32)
        + jnp.dot(xv, wr1bT[...], preferred_element_type=jnp.float32)
        + bl1b[...], 0.0)


def _mlp1(aggp, cntp, xp, wl1aT, wl1bT, wr1aT, wr1bT, bl1a, bl1b):
    grid = NPAD // BLK
    return pl.pallas_call(
        _mlp1_body,
        grid=(grid,),
        in_specs=[
            pl.BlockSpec((2, BLK, IN_CH), lambda i: (0, i, 0)),
            pl.BlockSpec((2, BLK, IN_CH), lambda i: (0, i, 0)),
            pl.BlockSpec((BLK, IN_CH), lambda i: (i, 0)),
            pl.BlockSpec((IN_CH, 128), lambda i: (0, 0)),
            pl.BlockSpec((IN_CH, 128), lambda i: (0, 0)),
            pl.BlockSpec((IN_CH, 128), lambda i: (0, 0)),
            pl.BlockSpec((IN_CH, 128), lambda i: (0, 0)),
            pl.BlockSpec((1, 128), lambda i: (0, 0)),
            pl.BlockSpec((1, 128), lambda i: (0, 0)),
        ],
        out_specs=[
            pl.BlockSpec((BLK, 128), lambda i: (i, 0)),
            pl.BlockSpec((BLK, 128), lambda i: (i, 0)),
            pl.BlockSpec((BLK, 8), lambda i: (i, 0)),
        ],
        out_shape=[
            jax.ShapeDtypeStruct((NPAD, 128), jnp.float32),
            jax.ShapeDtypeStruct((NPAD, 128), jnp.float32),
            jax.ShapeDtypeStruct((NPAD, 8), jnp.float32),
        ],
    )(aggp, cntp, xp, wl1aT, wl1bT, wr1aT, wr1bT, bl1a, bl1b)


def _mlp2_body(agg2p, rec8, h1a, h1b, wl2aT, wl2bT, wr2aT, wr2bT, bl2r,
               w1T, b1r, z1):
    rec = rec8[:, 0:1]
    aA = agg2p[0] * rec
    aB = agg2p[1] * rec
    h2 = jnp.maximum(
        jnp.dot(aA, wl2aT[...], preferred_element_type=jnp.float32)
        + jnp.dot(aB, wl2bT[...], preferred_element_type=jnp.float32)
        + jnp.dot(h1a[...], wr2aT[...], preferred_element_type=jnp.float32)
        + jnp.dot(h1b[...], wr2bT[...], preferred_element_type=jnp.float32)
        + bl2r[...], 0.0)
    z1[...] = jnp.dot(h2, w1T[...], preferred_element_type=jnp.float32) + b1r[...]


def _mlp2(agg2p, rec8, h1a, h1b, wl2aT, wl2bT, wr2aT, wr2bT, bl2r, w1T, b1r):
    grid = NPAD // BLK
    return pl.pallas_call(
        _mlp2_body,
        grid=(grid,),
        in_specs=[
            pl.BlockSpec((2, BLK, 128), lambda i: (0, i, 0)),
            pl.BlockSpec((BLK, 8), lambda i: (i, 0)),
            pl.BlockSpec((BLK, 128), lambda i: (i, 0)),
            pl.BlockSpec((BLK, 128), lambda i: (i, 0)),
            pl.BlockSpec((128, HID), lambda i: (0, 0)),
            pl.BlockSpec((128, HID), lambda i: (0, 0)),
            pl.BlockSpec((128, HID), lambda i: (0, 0)),
            pl.BlockSpec((128, HID), lambda i: (0, 0)),
            pl.BlockSpec((1, HID), lambda i: (0, 0)),
            pl.BlockSpec((HID, H1), lambda i: (0, 0)),
            pl.BlockSpec((1, H1), lambda i: (0, 0)),
        ],
        out_specs=pl.BlockSpec((BLK, H1), lambda i: (i, 0)),
        out_shape=jax.ShapeDtypeStruct((NPAD, H1), jnp.float32),
    )(agg2p, rec8, h1a, h1b, wl2aT, wl2bT, wr2aT, wr2bT, bl2r, w1T, b1r)


def _head_body(z1, g1r, be1r, w2T, b2r, g2r, be2r, w3T, b3r, out):
    z = z1[...]
    mask = (lax.broadcasted_iota(jnp.int32, (NPAD, 1), 0) < N).astype(jnp.float32)
    inv = 1.0 / N
    mu1 = jnp.sum(z * mask, axis=0, keepdims=True) * inv
    d1 = (z - mu1) * mask
    var1 = jnp.sum(d1 * d1, axis=0, keepdims=True) * inv
    a1 = jnp.maximum(g1r[...] * (z - mu1) * lax.rsqrt(var1 + EPS) + be1r[...], 0.0)
    z2 = jnp.dot(a1, w2T[...], preferred_element_type=jnp.float32) + b2r[...]
    mu2 = jnp.sum(z2 * mask, axis=0, keepdims=True) * inv
    d2 = (z2 - mu2) * mask
    var2 = jnp.sum(d2 * d2, axis=0, keepdims=True) * inv
    a2 = jnp.maximum(g2r[...] * (z2 - mu2) * lax.rsqrt(var2 + EPS) + be2r[...], 0.0)
    out[...] = jnp.dot(a2, w3T[...], preferred_element_type=jnp.float32) + b3r[...]


def _head(z1, g1r, be1r, w2T, b2r, g2r, be2r, w3T8, b3r8):
    return pl.pallas_call(
        _head_body,
        out_shape=jax.ShapeDtypeStruct((NPAD, 8), jnp.float32),
    )(z1, g1r, be1r, w2T, b2r, g2r, be2r, w3T8, b3r8)


def kernel(x, edge_index, Wl1, bl1, Wr1, Wl2, bl2, Wr2,
           W1, b1, g1, be1, W2, b2, g2, be2, W3, b3):
    f32 = jnp.float32
    xp = jnp.pad(x, ((0, NPAD - N), (0, 0)))

    ei = edge_index.astype(jnp.int32)
    src = ei[0].reshape(32, E_PER_TILE_1)
    dst = ei[1].reshape(32, E_PER_TILE_1)
    srcp = jnp.pad(src, ((0, 0), (0, E_PAD_1 - E_PER_TILE_1)))
    dstp = jnp.pad(dst, ((0, 0), (0, E_PAD_1 - E_PER_TILE_1)), constant_values=N)
    src1 = srcp.reshape(32, NCH1, CHUNK)
    dst1 = dstp.reshape(32, NCH1, CHUNK)
    src2 = srcp.reshape(16, NCH2, CHUNK)
    dst2 = dstp.reshape(16, NCH2, CHUNK)
    z128 = jnp.zeros((CHUNK, IN_CH), f32)
    o128 = jnp.ones((CHUNK, IN_CH), f32)

    cntp = _cnt()(dst1, z128, o128)
    aggp = _agg1()(xp, src1, dst1, z128)

    wl1T = Wl1.T
    wr1T = Wr1.T
    h1a, h1b, rec8 = _mlp1(aggp, cntp, xp,
                           wl1T[:, :128], wl1T[:, 128:],
                           wr1T[:, :128], wr1T[:, 128:],
                           bl1[:128].reshape(1, 128), bl1[128:].reshape(1, 128))

    agg2p = _agg2()(h1a, h1b, src2, dst2, z128)

    wl2T = Wl2.T
    wr2T = Wr2.T
    z1 = _mlp2(agg2p, rec8, h1a, h1b,
               wl2T[:128], wl2T[128:], wr2T[:128], wr2T[128:],
               bl2.reshape(1, HID), W1.T, b1.reshape(1, H1))

    w3T8 = jnp.broadcast_to(W3.T, (H2, 8))
    b3r8 = jnp.broadcast_to(b3.reshape(1, 1), (1, 8))
    out8 = _head(z1, g1.reshape(1, H1), be1.reshape(1, H1),
                 W2.T, b2.reshape(1, H2), g2.reshape(1, H2), be2.reshape(1, H2),
                 w3T8, b3r8)
    return out8[:N, 0]
